# baseline (speedup 1.0000x reference)
import functools

import jax
import jax.numpy as jnp
from jax import lax
from jax.experimental import pallas as pl
from jax.experimental.pallas import tpu as pltpu
from jax.experimental.pallas import tpu_sc as plsc

EMBED_DIM = 64
BATCH = 16384

_info = plsc.get_sparse_core_info()
_NC, _NS = _info.num_cores, _info.num_subcores
_NW = _NC * _NS
_B_PER_W = BATCH // _NW

_mesh = plsc.VectorSubcoreMesh(core_axis_name="c", subcore_axis_name="s")


@functools.partial(
    pl.kernel,
    mesh=_mesh,
    out_type=jax.ShapeDtypeStruct((BATCH, EMBED_DIM), jnp.float32),
    scratch_types=[
        pltpu.VMEM((_B_PER_W,), jnp.int32),
        pltpu.VMEM((_B_PER_W, EMBED_DIM), jnp.float32),
        pltpu.SemaphoreType.DMA,
    ],
    compiler_params=pltpu.CompilerParams(use_tc_tiling_on_sc=False),
)
def _gather_kernel(idx_hbm, table_hbm, out_hbm, idx_v, rows_v, sem):
    wid = lax.axis_index("s") * _NC + lax.axis_index("c")
    base = wid * _B_PER_W
    pltpu.sync_copy(idx_hbm.at[pl.ds(base, _B_PER_W)], idx_v)
    pltpu.async_copy(table_hbm.at[idx_v], rows_v, sem).wait()
    pltpu.sync_copy(rows_v, out_hbm.at[pl.ds(base, _B_PER_W)])


def kernel(indices, table):
    return _gather_kernel(indices.astype(jnp.int32), table)


# speedup vs baseline: 1.0969x; 1.0969x over previous
import functools

import jax
import jax.numpy as jnp
from jax import lax
from jax.experimental import pallas as pl
from jax.experimental.pallas import tpu as pltpu
from jax.experimental.pallas import tpu_sc as plsc

EMBED_DIM = 64
PAD_DIM = 128
NUM_ROWS = 100000
BATCH = 16384
LANES = 16
CHUNK = 128

_info = plsc.get_sparse_core_info()
_NC, _NS = _info.num_cores, _info.num_subcores
_NW = _NC * _NS
_B_PER_W = BATCH // _NW
_NCHUNKS = _B_PER_W // CHUNK

_mesh = plsc.VectorSubcoreMesh(core_axis_name="c", subcore_axis_name="s")


@functools.partial(
    pl.kernel,
    mesh=_mesh,
    out_type=jax.ShapeDtypeStruct((BATCH, EMBED_DIM), jnp.float32),
    scratch_types=[
        pltpu.VMEM((_B_PER_W,), jnp.int32),
        pltpu.VMEM((2, CHUNK, PAD_DIM), jnp.float32),
        pltpu.VMEM((CHUNK, EMBED_DIM), jnp.float32),
        pltpu.SemaphoreType.DMA,
        pltpu.SemaphoreType.DMA,
    ],
)
def _gather_kernel(idx_hbm, tab_hbm, out_hbm, idx_v, rows_v, stage_v,
                   sem0, sem1):
    wid = lax.axis_index("s") * _NC + lax.axis_index("c")
    base = wid * _B_PER_W
    pltpu.sync_copy(idx_hbm.at[pl.ds(base, _B_PER_W)], idx_v)

    sems = (sem0, sem1)

    def _gather(c):
        return pltpu.async_copy(tab_hbm.at[idx_v.at[pl.ds(c * CHUNK, CHUNK)]],
                                rows_v.at[c % 2], sems[c % 2])

    def _compact(c):
        def body(r, carry):
            for k in range(EMBED_DIM // LANES):
                sl = pl.ds(k * LANES, LANES)
                stage_v[r, sl] = rows_v[c % 2, r, sl]
            return carry
        lax.fori_loop(0, CHUNK, body, 0, unroll=4)

    pending = _gather(0)
    for c in range(_NCHUNKS):
        if c + 1 < _NCHUNKS:
            nxt = _gather(c + 1)
        pending.wait()
        _compact(c)
        pltpu.sync_copy(stage_v, out_hbm.at[pl.ds(base + c * CHUNK, CHUNK)])
        if c + 1 < _NCHUNKS:
            pending = nxt


def kernel(indices, table):
    tab_pad = jnp.pad(table, ((0, 0), (0, PAD_DIM - EMBED_DIM)))
    return _gather_kernel(indices.astype(jnp.int32), tab_pad)
